# baseline (speedup 1.0000x reference)
import dataclasses
import functools

import jax
import jax.numpy as jnp
from jax import lax
from jax.experimental import pallas as pl
from jax.experimental.pallas import tpu as pltpu
from jax.experimental.pallas import tpu_sc as plsc

_N = 10000
_E = 320000
_B = 16
_MAX_LEN = 8
_HID = 128
_LAYERS = 3

_NC = 2
_NS = 16
_NW = _NC * _NS
_CHUNK = 128
_CPW = 80
_SUPER = 2
_EPW = _CPW * _CHUNK
_EPAD = _NW * _EPW
_ACC_ROWS = 10016
_NSUPER = _CPW // _SUPER
_SROWS = _SUPER * _CHUNK


def _sc_gather_rows(table, idx3):
    d = table.shape[1]
    mesh = plsc.VectorSubcoreMesh(core_axis_name="c", subcore_axis_name="s")

    @functools.partial(
        pl.kernel,
        mesh=mesh,
        out_type=jax.ShapeDtypeStruct((_EPAD, d), jnp.float32),
        scratch_types=[
            pltpu.VMEM((_CPW, _CHUNK), jnp.int32),
            pltpu.VMEM((_CHUNK, d), jnp.float32),
            pltpu.VMEM((_CHUNK, d), jnp.float32),
            pltpu.VMEM_SHARED((table.shape[0], d), jnp.float32),
            pltpu.SemaphoreType.DMA,
            pltpu.SemaphoreType.DMA,
        ],
    )
    def gather_kernel(table_hbm, idx_hbm, out_hbm, idx_v, buf_a, buf_b,
                      tab_sh, sem_a, sem_b):
        sid = lax.axis_index("s")
        wid = lax.axis_index("c") * _NS + sid
        pltpu.sync_copy(idx_hbm.at[wid], idx_v)

        @pl.when(sid == 0)
        def _():
            pltpu.sync_copy(table_hbm, tab_sh)

        plsc.subcore_barrier()
        base0 = wid * _EPW
        ring = ((buf_a, sem_a), (buf_b, sem_b))

        def fire(c, buf, sem):
            pltpu.async_copy(tab_sh.at[idx_v.at[c]], buf, sem)

        def wait_g(buf, sem):
            pltpu.make_async_copy(
                tab_sh.at[idx_v.at[0]], buf, sem).wait()

        def wb(c, buf):
            pltpu.sync_copy(buf, out_hbm.at[pl.ds(base0 + c * _CHUNK, _CHUNK)])

        for k, (buf, sem) in enumerate(ring):
            fire(k, buf, sem)

        @pl.loop(0, _CPW, step=2)
        def _(c):
            for k, (buf, sem) in enumerate(ring):
                wait_g(buf, sem)
                wb(c + k, buf)

                @pl.when(c + k + 2 < _CPW)
                def _():
                    fire(c + k + 2, buf, sem)

    return gather_kernel(table, idx3)


def _sc_scatter_add2(ex, w, dst3, zeros):
    d = ex.shape[1]
    rows_per_sub = (_N // _NS) // 8 * 8
    tail_rows = _N - rows_per_sub * _NS
    mesh = plsc.VectorSubcoreMesh(core_axis_name="c", subcore_axis_name="s")
    out_sd = jax.ShapeDtypeStruct((_NC, _N, d), jnp.float32)

    @functools.partial(
        pl.kernel,
        mesh=mesh,
        out_type=(out_sd, out_sd),
        scratch_types=[
            pltpu.VMEM((_CPW, _CHUNK), jnp.int32),
            pltpu.VMEM((_CHUNK, d), jnp.float32),
            pltpu.VMEM((_CHUNK, d), jnp.float32),
            pltpu.VMEM_SHARED((_ACC_ROWS, d), jnp.float32),
            pltpu.SemaphoreType.DMA,
            pltpu.SemaphoreType.DMA,
        ],
    )
    def scatter_kernel(ex_hbm, w_hbm, dst_hbm, zeros_hbm, o1_hbm, o2_hbm,
                       idx_v, buf_a, buf_b, acc_sh, sem_a, sem_b):
        cid = lax.axis_index("c")
        sid = lax.axis_index("s")
        wid = cid * _NS + sid
        pltpu.sync_copy(dst_hbm.at[wid], idx_v)
        base0 = wid * _EPW

        for vals_hbm, out_hbm in ((ex_hbm, o1_hbm), (w_hbm, o2_hbm)):
            @pl.when(sid == 0)
            def _():
                pltpu.sync_copy(zeros_hbm, acc_sh)

            plsc.subcore_barrier()

            def vfire(c, buf, sem):
                pltpu.async_copy(
                    vals_hbm.at[pl.ds(base0 + c * _CHUNK, _CHUNK)], buf, sem)

            def vwait(buf, sem):
                pltpu.make_async_copy(
                    vals_hbm.at[pl.ds(base0, _CHUNK)], buf, sem).wait()

            def scat(c, buf):
                pltpu.sync_copy(buf, acc_sh.at[idx_v.at[c]], add=True)

            vfire(0, buf_a, sem_a)
            vfire(1, buf_b, sem_b)

            @pl.loop(0, _CPW, step=2)
            def _(c):
                vwait(buf_a, sem_a)
                scat(c, buf_a)

                @pl.when(c + 2 < _CPW)
                def _():
                    vfire(c + 2, buf_a, sem_a)

                vwait(buf_b, sem_b)
                scat(c + 1, buf_b)

                @pl.when(c + 3 < _CPW)
                def _():
                    vfire(c + 3, buf_b, sem_b)

            plsc.subcore_barrier()
            pltpu.sync_copy(
                acc_sh.at[pl.ds(sid * rows_per_sub, rows_per_sub)],
                out_hbm.at[cid, pl.ds(sid * rows_per_sub, rows_per_sub)],
            )
            if tail_rows:
                @pl.when(sid == _NS - 1)
                def _():
                    pltpu.sync_copy(
                        acc_sh.at[pl.ds(_NS * rows_per_sub, tail_rows)],
                        out_hbm.at[cid, pl.ds(_NS * rows_per_sub, tail_rows)],
                    )
            plsc.subcore_barrier()

    return scatter_kernel(ex, w, dst3, zeros)


def _sc_scatter_tab0(ex_tab, w_tab, pidx3, dst_flat, zeros):
    d = ex_tab.shape[1]
    rows_per_sub = (_N // _NS) // 8 * 8
    tail_rows = _N - rows_per_sub * _NS
    mesh = plsc.VectorSubcoreMesh(core_axis_name="c", subcore_axis_name="s")
    out_sd = jax.ShapeDtypeStruct((_NC, _N, d), jnp.float32)

    @functools.partial(
        pl.kernel,
        mesh=mesh,
        out_type=(out_sd, out_sd),
        scratch_types=[
            pltpu.VMEM((_CHUNK,), jnp.int32),
            pltpu.VMEM((_CHUNK,), jnp.int32),
            pltpu.VMEM((_CHUNK,), jnp.int32),
            pltpu.VMEM((_CHUNK,), jnp.int32),
            pltpu.VMEM((_CHUNK,), jnp.int32),
            pltpu.VMEM((_CHUNK,), jnp.int32),
            pltpu.VMEM((_CHUNK,), jnp.int32),
            pltpu.VMEM((_CHUNK,), jnp.int32),
            pltpu.VMEM((_CHUNK, d), jnp.float32),
            pltpu.VMEM((_CHUNK, d), jnp.float32),
            pltpu.VMEM_SHARED((_ACC_ROWS, d), jnp.float32),
            pltpu.VMEM_SHARED((100 * 16, d), jnp.float32),
            pltpu.SemaphoreType.DMA,
            pltpu.SemaphoreType.DMA,
            pltpu.SemaphoreType.DMA,
            pltpu.SemaphoreType.DMA,
            pltpu.SemaphoreType.DMA,
            pltpu.SemaphoreType.DMA,
        ],
    )
    def scatter_kernel(ex_hbm, w_hbm, pidx_hbm, dst_hbm, zeros_hbm, o1_hbm,
                       o2_hbm, p0, d0, p1, d1, p2, d2, p3, d3, buf_a, buf_b,
                       acc_sh, tab_sh, si0, si1, si2, si3, sv_a, sv_b):
        cid = lax.axis_index("c")
        sid = lax.axis_index("s")
        wid = cid * _NS + sid
        base0 = wid * _EPW
        islots = ((p0, d0, si0), (p1, d1, si1), (p2, d2, si2), (p3, d3, si3))

        def ifire(c, k):
            pbuf, dbuf, sem = islots[k]
            pltpu.async_copy(
                pidx_hbm.at[pl.ds(base0 + c * _CHUNK, _CHUNK)], pbuf, sem)
            pltpu.async_copy(
                dst_hbm.at[pl.ds(base0 + c * _CHUNK, _CHUNK)], dbuf, sem)

        def iwait(k):
            pbuf, dbuf, sem = islots[k]
            pltpu.make_async_copy(
                pidx_hbm.at[pl.ds(base0, _CHUNK)], pbuf, sem).wait()
            pltpu.make_async_copy(
                dst_hbm.at[pl.ds(base0, _CHUNK)], dbuf, sem).wait()

        for tab_hbm, out_hbm in ((ex_hbm, o1_hbm), (w_hbm, o2_hbm)):
            @pl.when(sid == 0)
            def _():
                pltpu.sync_copy(zeros_hbm, acc_sh)

            @pl.when(sid == 1)
            def _():
                pltpu.sync_copy(tab_hbm, tab_sh)

            plsc.subcore_barrier()

            def vfire(k, vbuf, sem):
                pltpu.async_copy(tab_sh.at[islots[k][0]], vbuf, sem)

            def vwait(vbuf, sem):
                pltpu.make_async_copy(tab_sh.at[p0], vbuf, sem).wait()

            def scat(k, vbuf):
                pltpu.sync_copy(vbuf, acc_sh.at[islots[k][1]], add=True)

            for k in range(4):
                ifire(k, k)
            iwait(0)
            vfire(0, buf_a, sv_a)

            @pl.loop(0, _CPW, step=4)
            def _(c):
                iwait(1)
                vfire(1, buf_b, sv_b)
                vwait(buf_a, sv_a)
                scat(0, buf_a)

                @pl.when(c + 4 < _CPW)
                def _():
                    ifire(c + 4, 0)

                iwait(2)
                vfire(2, buf_a, sv_a)
                vwait(buf_b, sv_b)
                scat(1, buf_b)

                @pl.when(c + 5 < _CPW)
                def _():
                    ifire(c + 5, 1)

                iwait(3)
                vfire(3, buf_b, sv_b)
                vwait(buf_a, sv_a)
                scat(2, buf_a)

                @pl.when(c + 6 < _CPW)
                def _():
                    ifire(c + 6, 2)

                @pl.when(c + 4 < _CPW)
                def _():
                    iwait(0)
                    vfire(0, buf_a, sv_a)

                vwait(buf_b, sv_b)
                scat(3, buf_b)

                @pl.when(c + 7 < _CPW)
                def _():
                    ifire(c + 7, 3)

            plsc.subcore_barrier()
            pltpu.sync_copy(
                acc_sh.at[pl.ds(sid * rows_per_sub, rows_per_sub)],
                out_hbm.at[cid, pl.ds(sid * rows_per_sub, rows_per_sub)],
            )
            if tail_rows:
                @pl.when(sid == _NS - 1)
                def _():
                    pltpu.sync_copy(
                        acc_sh.at[pl.ds(_NS * rows_per_sub, tail_rows)],
                        out_hbm.at[cid, pl.ds(_NS * rows_per_sub, tail_rows)],
                    )
            plsc.subcore_barrier()

    return scatter_kernel(ex_tab, w_tab, pidx3, dst_flat, zeros)


def _tc_tab0(hv, ntab_pad, etab_pad, bn_w_row, bn_b_row, beta_row):
    def body(hv_ref, ntab_ref, etab_ref, w_ref, b_ref, beta_ref,
             hv1_ref, ex_ref, wt_ref):
        x = hv_ref[...]
        mu = jnp.mean(x, axis=0, keepdims=True)
        dxc = x - mu
        var = jnp.mean(dxc * dxc, axis=0, keepdims=True)
        scale = lax.rsqrt(var + 1e-5)
        y = dxc * scale * w_ref[...] + b_ref[...]
        y = jnp.maximum(y, 0.0)
        hv1_ref[...] = y
        tab1 = (ntab_ref[...] - mu) * scale * w_ref[...] + b_ref[...]
        tab1 = jnp.maximum(tab1, 0.0)
        emax = jnp.max(etab_ref[...], axis=0, keepdims=True)
        mb = jnp.max(tab1, axis=0, keepdims=True) + emax + 1e-7
        m_row = jnp.maximum(beta_ref[...] * mb, 0.0)
        m3 = jnp.maximum(
            tab1[:100, None, :] + etab_ref[...][None, :, :], 0.0)
        m3 = m3 + 1e-7
        mm = m3.reshape(100 * 16, _HID)
        ex = jnp.exp(mm * beta_ref[...] - m_row)
        ex_ref[...] = ex
        wt_ref[...] = mm * ex

    return pl.pallas_call(
        body,
        out_shape=(
            jax.ShapeDtypeStruct((_N, _HID), jnp.float32),
            jax.ShapeDtypeStruct((100 * 16, _HID), jnp.float32),
            jax.ShapeDtypeStruct((100 * 16, _HID), jnp.float32),
        ),
    )(hv, ntab_pad, etab_pad, bn_w_row, bn_b_row, beta_row)


def _sc_pidx(node_cat_arr, src_flat, cat_flat):
    mesh = plsc.VectorSubcoreMesh(core_axis_name="c", subcore_axis_name="s")
    cp = pltpu.CompilerParams()
    if "needs_layout_passes" in pltpu.CompilerParams.__dataclass_fields__:
        cp = dataclasses.replace(cp, needs_layout_passes=False)

    @functools.partial(
        pl.kernel,
        mesh=mesh,
        out_type=jax.ShapeDtypeStruct((_EPAD,), jnp.int32),
        compiler_params=cp,
        scratch_types=[
            pltpu.VMEM((_N,), jnp.int32),
            pltpu.VMEM((_EPW,), jnp.int32),
            pltpu.VMEM((_EPW,), jnp.int32),
            pltpu.VMEM((_EPW,), jnp.int32),
        ],
    )
    def pidx_kernel(nc_hbm, src_hbm, cat_hbm, out_hbm, nc_v, src_v, cat_v,
                    o_v):
        wid = lax.axis_index("c") * _NS + lax.axis_index("s")
        base = wid * _EPW
        pltpu.sync_copy(nc_hbm, nc_v)
        pltpu.sync_copy(src_hbm.at[pl.ds(base, _EPW)], src_v)
        pltpu.sync_copy(cat_hbm.at[pl.ds(base, _EPW)], cat_v)

        @pl.loop(0, _EPW, step=16)
        def _(i):
            sv = src_v[pl.ds(i, 16)]
            nc = plsc.load_gather(nc_v, [sv])
            o_v[pl.ds(i, 16)] = nc * 16 + cat_v[pl.ds(i, 16)]

        pltpu.sync_copy(o_v, out_hbm.at[pl.ds(base, _EPW)])

    return pidx_kernel(node_cat_arr, src_flat, cat_flat)


def _tc_embed(cat_col, table_pad):
    n = cat_col.shape[0]
    v = table_pad.shape[0]

    def body(cat_ref, tab_ref, out_ref):
        io = lax.broadcasted_iota(jnp.int32, (n, v), 1).astype(jnp.float32)
        oh = (cat_ref[...] == io).astype(jnp.float32)
        out_ref[...] = jnp.dot(oh, tab_ref[...],
                               preferred_element_type=jnp.float32)

    return pl.pallas_call(
        body,
        out_shape=jax.ShapeDtypeStruct((n, _HID), jnp.float32),
    )(cat_col, table_pad)


def _tc_bn_relu(hv, bn_w_row, bn_b_row, beta_row, edge_emb):
    def body(hv_ref, w_ref, b_ref, beta_ref, emb_ref, hv1_ref, m_ref):
        x = hv_ref[...]
        mu = jnp.mean(x, axis=0, keepdims=True)
        dxc = x - mu
        var = jnp.mean(dxc * dxc, axis=0, keepdims=True)
        y = dxc * lax.rsqrt(var + 1e-5) * w_ref[...] + b_ref[...]
        y = jnp.maximum(y, 0.0)
        hv1_ref[...] = y
        emax = jnp.max(emb_ref[...], axis=0, keepdims=True)
        mb = jnp.max(y, axis=0, keepdims=True) + emax + 1e-7
        m_ref[...] = jnp.maximum(beta_ref[...] * mb, 0.0)

    return pl.pallas_call(
        body,
        out_shape=(
            jax.ShapeDtypeStruct((_N, _HID), jnp.float32),
            jax.ShapeDtypeStruct((1, _HID), jnp.float32),
        ),
    )(hv, bn_w_row, bn_b_row, beta_row, edge_emb)


def _tc_edge_elem(g, cat_col, m_row, beta_row, etab_pad):
    eb = 4096
    grid = _EPAD // eb

    def body(g_ref, cat_ref, m_ref, beta_ref, etab_ref, ex_ref, w_ref):
        io = lax.broadcasted_iota(jnp.int32, (eb, 16), 1).astype(jnp.float32)
        oh = (cat_ref[...] == io).astype(jnp.float32)
        he = jnp.dot(oh, etab_ref[...], preferred_element_type=jnp.float32)
        m = jnp.maximum(g_ref[...] + he, 0.0) + 1e-7
        ex = jnp.exp(m * beta_ref[...] - m_ref[...])
        ex_ref[...] = ex
        w_ref[...] = m * ex

    return pl.pallas_call(
        body,
        grid=(grid,),
        in_specs=[
            pl.BlockSpec((eb, _HID), lambda i: (i, 0)),
            pl.BlockSpec((eb, 1), lambda i: (i, 0)),
            pl.BlockSpec((1, _HID), lambda i: (0, 0)),
            pl.BlockSpec((1, _HID), lambda i: (0, 0)),
            pl.BlockSpec((16, _HID), lambda i: (0, 0)),
        ],
        out_specs=[
            pl.BlockSpec((eb, _HID), lambda i: (i, 0)),
            pl.BlockSpec((eb, _HID), lambda i: (i, 0)),
        ],
        out_shape=(
            jax.ShapeDtypeStruct((_EPAD, _HID), jnp.float32),
            jax.ShapeDtypeStruct((_EPAD, _HID), jnp.float32),
        ),
    )(g, cat_col, m_row, beta_row, etab_pad)


def _tc_combine_linear(s_part, w_part, hv1, hv_prev, lin_wt, lin_b_row):
    def body(sp_ref, wp_ref, hv1_ref, hvp_ref, wt_ref, b_ref, out_ref):
        ssum = sp_ref[0] + sp_ref[1]
        wsum = wp_ref[0] + wp_ref[1]
        agg = wsum / (ssum + 1e-16)
        feats = hv1_ref[...] + agg
        conv = jnp.dot(feats, wt_ref[...],
                       preferred_element_type=jnp.float32) + b_ref[...]
        out_ref[...] = conv + hvp_ref[...]

    return pl.pallas_call(
        body,
        out_shape=jax.ShapeDtypeStruct((_N, _HID), jnp.float32),
    )(s_part, w_part, hv1, hv_prev, lin_wt, lin_b_row)


def _tc_head(hv, gid_row, grp_row, wih_f, whh_f, bsum_f, wih_b, whh_b,
             bsum_b, wo1, wo2, bout_row):
    n_seg = _B * _MAX_LEN

    def body(hv_ref, gid_ref, grp_ref, wihf_ref, whhf_ref, bf_ref,
             wihb_ref, whhb_ref, bb_ref, wo1_ref, wo2_ref, bo_ref, out_ref):
        hv_v = hv_ref[...]
        comb = gid_ref[...] * _MAX_LEN + grp_ref[...]
        seg_iota = lax.broadcasted_iota(jnp.int32, (n_seg, _N), 0)
        oht = (seg_iota == comb).astype(jnp.float32)
        gsum = jnp.dot(oht, hv_v, preferred_element_type=jnp.float32)
        cnt = jnp.sum(oht, axis=1, keepdims=True)
        means = jnp.where(cnt > 0.0, gsum / jnp.maximum(cnt, 1.0), 0.0)
        cntpos = (cnt > 0.0).astype(jnp.float32)

        row_iota = lax.broadcasted_iota(jnp.int32, (_B, n_seg), 0)
        col_iota = lax.broadcasted_iota(jnp.int32, (_B, n_seg), 1)

        def step_sel(t):
            return (col_iota == row_iota * _MAX_LEN + t).astype(jnp.float32)

        acc = jnp.zeros((_B, _HID), dtype=jnp.float32)
        h = jnp.zeros((_B, _HID), dtype=jnp.float32)
        for t in range(_MAX_LEN):
            sel = step_sel(t)
            x = jnp.dot(sel, means, preferred_element_type=jnp.float32)
            h = jnp.tanh(
                jnp.dot(x, wihf_ref[...], preferred_element_type=jnp.float32)
                + jnp.dot(h, whhf_ref[...], preferred_element_type=jnp.float32)
                + bf_ref[...])
            mask = jnp.dot(sel, cntpos, preferred_element_type=jnp.float32)
            acc = acc + 0.5 * h * mask
        h = jnp.zeros((_B, _HID), dtype=jnp.float32)
        for t in range(_MAX_LEN - 1, -1, -1):
            sel = step_sel(t)
            x = jnp.dot(sel, means, preferred_element_type=jnp.float32)
            h = jnp.tanh(
                jnp.dot(x, wihb_ref[...], preferred_element_type=jnp.float32)
                + jnp.dot(h, whhb_ref[...], preferred_element_type=jnp.float32)
                + bb_ref[...])
            mask = jnp.dot(sel, cntpos, preferred_element_type=jnp.float32)
            acc = acc + 0.5 * h * mask

        gid_iota = lax.broadcasted_iota(jnp.int32, (_B, _N), 0)
        ohg = (gid_iota == gid_ref[...]).astype(jnp.float32)
        gp = jnp.dot(ohg, hv_v, preferred_element_type=jnp.float32)
        gcnt = jnp.sum(ohg, axis=1, keepdims=True)
        gpool = gp / jnp.maximum(gcnt, 1.0)

        out_ref[...] = (
            jnp.dot(acc, wo1_ref[...], preferred_element_type=jnp.float32)
            + jnp.dot(gpool, wo2_ref[...], preferred_element_type=jnp.float32)
            + bo_ref[...])

    return pl.pallas_call(
        body,
        out_shape=jax.ShapeDtypeStruct((_B, _HID), jnp.float32),
    )(hv, gid_row, grp_row, wih_f, whh_f, bsum_f, wih_b, whh_b, bsum_b,
      wo1, wo2, bout_row)


def kernel(edge_index, node_cat, edge_cat, graph_ids, group_ids, node_emb,
           edge_emb, bn_w, bn_b, lin_W, lin_b, beta, W_ih, W_hh, b_ih, b_hh,
           W_out, b_out):
    src = edge_index[0]
    dst = edge_index[1]
    npad = _EPAD - _E

    src_pad = jnp.concatenate([src, jnp.zeros((npad,), jnp.int32)])
    dst_pad = jnp.concatenate([dst, jnp.full((npad,), _N, jnp.int32)])
    src3 = src_pad.reshape(_NW, _CPW, _CHUNK)
    dst3 = dst_pad.reshape(_NW, _CPW, _CHUNK)
    cat_col = jnp.concatenate(
        [edge_cat, jnp.zeros((npad,), jnp.int32)]
    ).astype(jnp.float32).reshape(_EPAD, 1)

    zeros_acc = jnp.zeros((_ACC_ROWS, _HID), dtype=jnp.float32)
    ntab_pad = jnp.concatenate(
        [node_emb, jnp.zeros((128 - node_emb.shape[0], _HID))]).astype(
            jnp.float32)
    etab_pad = jnp.concatenate(
        [edge_emb, jnp.zeros((16 - edge_emb.shape[0], _HID))]).astype(
            jnp.float32)

    hv = _tc_embed(node_cat.astype(jnp.float32).reshape(_N, 1), ntab_pad)

    for l in range(_LAYERS):
        bn_w_row = bn_w[l].reshape(1, _HID)
        bn_b_row = bn_b[l].reshape(1, _HID)
        beta_row = jnp.broadcast_to(beta[l], (1, _HID)).astype(jnp.float32)
        hv1, m_row = _tc_bn_relu(hv, bn_w_row, bn_b_row, beta_row, edge_emb)
        g = _sc_gather_rows(hv1, src3)
        ex, w = _tc_edge_elem(g, cat_col, m_row, beta_row, etab_pad)
        s_part, w_part = _sc_scatter_add2(ex, w, dst3, zeros_acc)
        hv = _tc_combine_linear(s_part, w_part, hv1, hv,
                                lin_W[l].T, lin_b[l].reshape(1, _HID))

    gid_row = graph_ids.reshape(1, _N)
    grp_row = group_ids.reshape(1, _N)
    bsum_f = (b_ih[0] + b_hh[0]).reshape(1, _HID)
    bsum_b = (b_ih[1] + b_hh[1]).reshape(1, _HID)
    return _tc_head(hv, gid_row, grp_row,
                    W_ih[0].T, W_hh[0].T, bsum_f,
                    W_ih[1].T, W_hh[1].T, bsum_b,
                    W_out[:, :_HID].T, W_out[:, _HID:].T,
                    b_out.reshape(1, _HID))


# speedup vs baseline: 1.1345x; 1.1345x over previous
import dataclasses
import functools

import jax
import jax.numpy as jnp
from jax import lax
from jax.experimental import pallas as pl
from jax.experimental.pallas import tpu as pltpu
from jax.experimental.pallas import tpu_sc as plsc

_N = 10000
_E = 320000
_B = 16
_MAX_LEN = 8
_HID = 128
_LAYERS = 3

_NC = 2
_NS = 16
_NW = _NC * _NS
_CHUNK = 128
_CPW = 80
_SUPER = 2
_EPW = _CPW * _CHUNK
_EPAD = _NW * _EPW
_ACC_ROWS = 10016
_NSUPER = _CPW // _SUPER
_SROWS = _SUPER * _CHUNK


def _sc_gather_rows(table, idx3):
    d = table.shape[1]
    mesh = plsc.VectorSubcoreMesh(core_axis_name="c", subcore_axis_name="s")

    @functools.partial(
        pl.kernel,
        mesh=mesh,
        out_type=jax.ShapeDtypeStruct((_EPAD, d), jnp.float32),
        scratch_types=[
            pltpu.VMEM((_CPW, _CHUNK), jnp.int32),
            pltpu.VMEM((_CHUNK, d), jnp.float32),
            pltpu.VMEM((_CHUNK, d), jnp.float32),
            pltpu.VMEM_SHARED((table.shape[0], d), jnp.float32),
            pltpu.SemaphoreType.DMA,
            pltpu.SemaphoreType.DMA,
        ],
    )
    def gather_kernel(table_hbm, idx_hbm, out_hbm, idx_v, buf_a, buf_b,
                      tab_sh, sem_a, sem_b):
        sid = lax.axis_index("s")
        wid = lax.axis_index("c") * _NS + sid
        pltpu.sync_copy(idx_hbm.at[wid], idx_v)

        @pl.when(sid == 0)
        def _():
            pltpu.sync_copy(table_hbm, tab_sh)

        plsc.subcore_barrier()
        base0 = wid * _EPW
        ring = ((buf_a, sem_a), (buf_b, sem_b))

        def fire(c, buf, sem):
            pltpu.async_copy(tab_sh.at[idx_v.at[c]], buf, sem)

        def wait_g(buf, sem):
            pltpu.make_async_copy(
                tab_sh.at[idx_v.at[0]], buf, sem).wait()

        def wb(c, buf):
            pltpu.sync_copy(buf, out_hbm.at[pl.ds(base0 + c * _CHUNK, _CHUNK)])

        for k, (buf, sem) in enumerate(ring):
            fire(k, buf, sem)

        @pl.loop(0, _CPW, step=2)
        def _(c):
            for k, (buf, sem) in enumerate(ring):
                wait_g(buf, sem)
                wb(c + k, buf)

                @pl.when(c + k + 2 < _CPW)
                def _():
                    fire(c + k + 2, buf, sem)

    return gather_kernel(table, idx3)


def _sc_scatter_add2(ex, w, dst3, zeros):
    d = ex.shape[1]
    rows_per_sub = (_N // _NS) // 8 * 8
    tail_rows = _N - rows_per_sub * _NS
    mesh = plsc.VectorSubcoreMesh(core_axis_name="c", subcore_axis_name="s")
    out_sd = jax.ShapeDtypeStruct((_NC, _N, d), jnp.float32)

    @functools.partial(
        pl.kernel,
        mesh=mesh,
        out_type=(out_sd, out_sd),
        scratch_types=[
            pltpu.VMEM((_CPW, _CHUNK), jnp.int32),
            pltpu.VMEM((_CHUNK, d), jnp.float32),
            pltpu.VMEM((_CHUNK, d), jnp.float32),
            pltpu.VMEM_SHARED((_ACC_ROWS, d), jnp.float32),
            pltpu.SemaphoreType.DMA,
            pltpu.SemaphoreType.DMA,
        ],
    )
    def scatter_kernel(ex_hbm, w_hbm, dst_hbm, zeros_hbm, o1_hbm, o2_hbm,
                       idx_v, buf_a, buf_b, acc_sh, sem_a, sem_b):
        cid = lax.axis_index("c")
        sid = lax.axis_index("s")
        wid = cid * _NS + sid
        pltpu.sync_copy(dst_hbm.at[wid], idx_v)
        base0 = wid * _EPW

        for vals_hbm, out_hbm in ((ex_hbm, o1_hbm), (w_hbm, o2_hbm)):
            @pl.when(sid == 0)
            def _():
                pltpu.sync_copy(zeros_hbm, acc_sh)

            plsc.subcore_barrier()

            def vfire(c, buf, sem):
                pltpu.async_copy(
                    vals_hbm.at[pl.ds(base0 + c * _CHUNK, _CHUNK)], buf, sem)

            def vwait(buf, sem):
                pltpu.make_async_copy(
                    vals_hbm.at[pl.ds(base0, _CHUNK)], buf, sem).wait()

            def scat(c, buf):
                pltpu.sync_copy(buf, acc_sh.at[idx_v.at[c]], add=True)

            vfire(0, buf_a, sem_a)
            vfire(1, buf_b, sem_b)

            @pl.loop(0, _CPW, step=2)
            def _(c):
                vwait(buf_a, sem_a)
                scat(c, buf_a)

                @pl.when(c + 2 < _CPW)
                def _():
                    vfire(c + 2, buf_a, sem_a)

                vwait(buf_b, sem_b)
                scat(c + 1, buf_b)

                @pl.when(c + 3 < _CPW)
                def _():
                    vfire(c + 3, buf_b, sem_b)

            plsc.subcore_barrier()
            pltpu.sync_copy(
                acc_sh.at[pl.ds(sid * rows_per_sub, rows_per_sub)],
                out_hbm.at[cid, pl.ds(sid * rows_per_sub, rows_per_sub)],
            )
            if tail_rows:
                @pl.when(sid == _NS - 1)
                def _():
                    pltpu.sync_copy(
                        acc_sh.at[pl.ds(_NS * rows_per_sub, tail_rows)],
                        out_hbm.at[cid, pl.ds(_NS * rows_per_sub, tail_rows)],
                    )
            plsc.subcore_barrier()

    return scatter_kernel(ex, w, dst3, zeros)


def _sc_scatter_tab0(ex_tab, w_tab, pidx3, dst_flat, zeros):
    d = ex_tab.shape[1]
    rows_per_sub = (_N // _NS) // 8 * 8
    tail_rows = _N - rows_per_sub * _NS
    mesh = plsc.VectorSubcoreMesh(core_axis_name="c", subcore_axis_name="s")
    out_sd = jax.ShapeDtypeStruct((_NC, _N, d), jnp.float32)

    @functools.partial(
        pl.kernel,
        mesh=mesh,
        out_type=(out_sd, out_sd),
        scratch_types=[
            pltpu.VMEM((_CHUNK,), jnp.int32),
            pltpu.VMEM((_CHUNK,), jnp.int32),
            pltpu.VMEM((_CHUNK,), jnp.int32),
            pltpu.VMEM((_CHUNK,), jnp.int32),
            pltpu.VMEM((_CHUNK,), jnp.int32),
            pltpu.VMEM((_CHUNK,), jnp.int32),
            pltpu.VMEM((_CHUNK,), jnp.int32),
            pltpu.VMEM((_CHUNK,), jnp.int32),
            pltpu.VMEM((_CHUNK, d), jnp.float32),
            pltpu.VMEM((_CHUNK, d), jnp.float32),
            pltpu.VMEM_SHARED((_ACC_ROWS, d), jnp.float32),
            pltpu.VMEM_SHARED((100 * 16, d), jnp.float32),
            pltpu.SemaphoreType.DMA,
            pltpu.SemaphoreType.DMA,
            pltpu.SemaphoreType.DMA,
            pltpu.SemaphoreType.DMA,
            pltpu.SemaphoreType.DMA,
            pltpu.SemaphoreType.DMA,
        ],
    )
    def scatter_kernel(ex_hbm, w_hbm, pidx_hbm, dst_hbm, zeros_hbm, o1_hbm,
                       o2_hbm, p0, d0, p1, d1, p2, d2, p3, d3, buf_a, buf_b,
                       acc_sh, tab_sh, si0, si1, si2, si3, sv_a, sv_b):
        cid = lax.axis_index("c")
        sid = lax.axis_index("s")
        wid = cid * _NS + sid
        base0 = wid * _EPW
        islots = ((p0, d0, si0), (p1, d1, si1), (p2, d2, si2), (p3, d3, si3))

        def ifire(c, k):
            pbuf, dbuf, sem = islots[k]
            pltpu.async_copy(
                pidx_hbm.at[pl.ds(base0 + c * _CHUNK, _CHUNK)], pbuf, sem)
            pltpu.async_copy(
                dst_hbm.at[pl.ds(base0 + c * _CHUNK, _CHUNK)], dbuf, sem)

        def iwait(k):
            pbuf, dbuf, sem = islots[k]
            pltpu.make_async_copy(
                pidx_hbm.at[pl.ds(base0, _CHUNK)], pbuf, sem).wait()
            pltpu.make_async_copy(
                dst_hbm.at[pl.ds(base0, _CHUNK)], dbuf, sem).wait()

        for tab_hbm, out_hbm in ((ex_hbm, o1_hbm), (w_hbm, o2_hbm)):
            @pl.when(sid == 0)
            def _():
                pltpu.sync_copy(zeros_hbm, acc_sh)

            @pl.when(sid == 1)
            def _():
                pltpu.sync_copy(tab_hbm, tab_sh)

            plsc.subcore_barrier()

            def vfire(k, vbuf, sem):
                pltpu.async_copy(tab_sh.at[islots[k][0]], vbuf, sem)

            def vwait(vbuf, sem):
                pltpu.make_async_copy(tab_sh.at[p0], vbuf, sem).wait()

            def scat(k, vbuf):
                pltpu.sync_copy(vbuf, acc_sh.at[islots[k][1]], add=True)

            for k in range(4):
                ifire(k, k)
            iwait(0)
            vfire(0, buf_a, sv_a)

            @pl.loop(0, _CPW, step=4)
            def _(c):
                iwait(1)
                vfire(1, buf_b, sv_b)
                vwait(buf_a, sv_a)
                scat(0, buf_a)

                @pl.when(c + 4 < _CPW)
                def _():
                    ifire(c + 4, 0)

                iwait(2)
                vfire(2, buf_a, sv_a)
                vwait(buf_b, sv_b)
                scat(1, buf_b)

                @pl.when(c + 5 < _CPW)
                def _():
                    ifire(c + 5, 1)

                iwait(3)
                vfire(3, buf_b, sv_b)
                vwait(buf_a, sv_a)
                scat(2, buf_a)

                @pl.when(c + 6 < _CPW)
                def _():
                    ifire(c + 6, 2)

                @pl.when(c + 4 < _CPW)
                def _():
                    iwait(0)
                    vfire(0, buf_a, sv_a)

                vwait(buf_b, sv_b)
                scat(3, buf_b)

                @pl.when(c + 7 < _CPW)
                def _():
                    ifire(c + 7, 3)

            plsc.subcore_barrier()
            pltpu.sync_copy(
                acc_sh.at[pl.ds(sid * rows_per_sub, rows_per_sub)],
                out_hbm.at[cid, pl.ds(sid * rows_per_sub, rows_per_sub)],
            )
            if tail_rows:
                @pl.when(sid == _NS - 1)
                def _():
                    pltpu.sync_copy(
                        acc_sh.at[pl.ds(_NS * rows_per_sub, tail_rows)],
                        out_hbm.at[cid, pl.ds(_NS * rows_per_sub, tail_rows)],
                    )
            plsc.subcore_barrier()

    return scatter_kernel(ex_tab, w_tab, pidx3, dst_flat, zeros)


def _tc_tab0(hv, ntab_pad, etab_pad, bn_w_row, bn_b_row, beta_row):
    def body(hv_ref, ntab_ref, etab_ref, w_ref, b_ref, beta_ref,
             hv1_ref, ex_ref, wt_ref):
        x = hv_ref[...]
        mu = jnp.mean(x, axis=0, keepdims=True)
        dxc = x - mu
        var = jnp.mean(dxc * dxc, axis=0, keepdims=True)
        scale = lax.rsqrt(var + 1e-5)
        y = dxc * scale * w_ref[...] + b_ref[...]
        y = jnp.maximum(y, 0.0)
        hv1_ref[...] = y
        tab1 = (ntab_ref[...] - mu) * scale * w_ref[...] + b_ref[...]
        tab1 = jnp.maximum(tab1, 0.0)
        emax = jnp.max(etab_ref[...], axis=0, keepdims=True)
        mb = jnp.max(tab1, axis=0, keepdims=True) + emax + 1e-7
        m_row = jnp.maximum(beta_ref[...] * mb, 0.0)
        m3 = jnp.maximum(
            tab1[:100, None, :] + etab_ref[...][None, :, :], 0.0)
        m3 = m3 + 1e-7
        mm = m3.reshape(100 * 16, _HID)
        ex = jnp.exp(mm * beta_ref[...] - m_row)
        ex_ref[...] = ex
        wt_ref[...] = mm * ex

    return pl.pallas_call(
        body,
        out_shape=(
            jax.ShapeDtypeStruct((_N, _HID), jnp.float32),
            jax.ShapeDtypeStruct((100 * 16, _HID), jnp.float32),
            jax.ShapeDtypeStruct((100 * 16, _HID), jnp.float32),
        ),
    )(hv, ntab_pad, etab_pad, bn_w_row, bn_b_row, beta_row)


def _sc_pidx(node_cat_arr, src_flat, cat_flat):
    mesh = plsc.VectorSubcoreMesh(core_axis_name="c", subcore_axis_name="s")
    cp = pltpu.CompilerParams()
    if "needs_layout_passes" in pltpu.CompilerParams.__dataclass_fields__:
        cp = dataclasses.replace(cp, needs_layout_passes=False)

    @functools.partial(
        pl.kernel,
        mesh=mesh,
        out_type=jax.ShapeDtypeStruct((_EPAD,), jnp.int32),
        compiler_params=cp,
        scratch_types=[
            pltpu.VMEM((_N,), jnp.int32),
            pltpu.VMEM((_EPW,), jnp.int32),
            pltpu.VMEM((_EPW,), jnp.int32),
            pltpu.VMEM((_EPW,), jnp.int32),
        ],
    )
    def pidx_kernel(nc_hbm, src_hbm, cat_hbm, out_hbm, nc_v, src_v, cat_v,
                    o_v):
        wid = lax.axis_index("c") * _NS + lax.axis_index("s")
        base = wid * _EPW
        pltpu.sync_copy(nc_hbm, nc_v)
        pltpu.sync_copy(src_hbm.at[pl.ds(base, _EPW)], src_v)
        pltpu.sync_copy(cat_hbm.at[pl.ds(base, _EPW)], cat_v)

        @pl.loop(0, _EPW, step=16)
        def _(i):
            sv = src_v[pl.ds(i, 16)]
            nc = plsc.load_gather(nc_v, [sv])
            o_v[pl.ds(i, 16)] = nc * 16 + cat_v[pl.ds(i, 16)]

        pltpu.sync_copy(o_v, out_hbm.at[pl.ds(base, _EPW)])

    return pidx_kernel(node_cat_arr, src_flat, cat_flat)


def _tc_embed(cat_col, table_pad):
    n = cat_col.shape[0]
    v = table_pad.shape[0]

    def body(cat_ref, tab_ref, out_ref):
        io = lax.broadcasted_iota(jnp.int32, (n, v), 1).astype(jnp.float32)
        oh = (cat_ref[...] == io).astype(jnp.float32)
        out_ref[...] = jnp.dot(oh, tab_ref[...],
                               preferred_element_type=jnp.float32)

    return pl.pallas_call(
        body,
        out_shape=jax.ShapeDtypeStruct((n, _HID), jnp.float32),
    )(cat_col, table_pad)


def _tc_bn_relu(hv, bn_w_row, bn_b_row, beta_row, edge_emb):
    def body(hv_ref, w_ref, b_ref, beta_ref, emb_ref, hv1_ref, m_ref):
        x = hv_ref[...]
        mu = jnp.mean(x, axis=0, keepdims=True)
        dxc = x - mu
        var = jnp.mean(dxc * dxc, axis=0, keepdims=True)
        y = dxc * lax.rsqrt(var + 1e-5) * w_ref[...] + b_ref[...]
        y = jnp.maximum(y, 0.0)
        hv1_ref[...] = y
        emax = jnp.max(emb_ref[...], axis=0, keepdims=True)
        mb = jnp.max(y, axis=0, keepdims=True) + emax + 1e-7
        m_ref[...] = jnp.maximum(beta_ref[...] * mb, 0.0)

    return pl.pallas_call(
        body,
        out_shape=(
            jax.ShapeDtypeStruct((_N, _HID), jnp.float32),
            jax.ShapeDtypeStruct((1, _HID), jnp.float32),
        ),
    )(hv, bn_w_row, bn_b_row, beta_row, edge_emb)


def _tc_edge_elem(g, cat_col, m_row, beta_row, etab_pad):
    eb = 4096
    grid = _EPAD // eb

    def body(g_ref, cat_ref, m_ref, beta_ref, etab_ref, ex_ref, w_ref):
        io = lax.broadcasted_iota(jnp.int32, (eb, 16), 1).astype(jnp.float32)
        oh = (cat_ref[...] == io).astype(jnp.float32)
        he = jnp.dot(oh, etab_ref[...], preferred_element_type=jnp.float32)
        m = jnp.maximum(g_ref[...] + he, 0.0) + 1e-7
        ex = jnp.exp(m * beta_ref[...] - m_ref[...])
        ex_ref[...] = ex
        w_ref[...] = m * ex

    return pl.pallas_call(
        body,
        grid=(grid,),
        in_specs=[
            pl.BlockSpec((eb, _HID), lambda i: (i, 0)),
            pl.BlockSpec((eb, 1), lambda i: (i, 0)),
            pl.BlockSpec((1, _HID), lambda i: (0, 0)),
            pl.BlockSpec((1, _HID), lambda i: (0, 0)),
            pl.BlockSpec((16, _HID), lambda i: (0, 0)),
        ],
        out_specs=[
            pl.BlockSpec((eb, _HID), lambda i: (i, 0)),
            pl.BlockSpec((eb, _HID), lambda i: (i, 0)),
        ],
        out_shape=(
            jax.ShapeDtypeStruct((_EPAD, _HID), jnp.float32),
            jax.ShapeDtypeStruct((_EPAD, _HID), jnp.float32),
        ),
    )(g, cat_col, m_row, beta_row, etab_pad)


def _tc_combine_linear(s_part, w_part, hv1, hv_prev, lin_wt, lin_b_row):
    def body(sp_ref, wp_ref, hv1_ref, hvp_ref, wt_ref, b_ref, out_ref):
        ssum = sp_ref[0] + sp_ref[1]
        wsum = wp_ref[0] + wp_ref[1]
        agg = wsum / (ssum + 1e-16)
        feats = hv1_ref[...] + agg
        conv = jnp.dot(feats, wt_ref[...],
                       preferred_element_type=jnp.float32) + b_ref[...]
        out_ref[...] = conv + hvp_ref[...]

    return pl.pallas_call(
        body,
        out_shape=jax.ShapeDtypeStruct((_N, _HID), jnp.float32),
    )(s_part, w_part, hv1, hv_prev, lin_wt, lin_b_row)


def _tc_head(hv, gid_row, grp_row, wih_f, whh_f, bsum_f, wih_b, whh_b,
             bsum_b, wo1, wo2, bout_row):
    n_seg = _B * _MAX_LEN

    def body(hv_ref, gid_ref, grp_ref, wihf_ref, whhf_ref, bf_ref,
             wihb_ref, whhb_ref, bb_ref, wo1_ref, wo2_ref, bo_ref, out_ref):
        hv_v = hv_ref[...]
        comb = gid_ref[...] * _MAX_LEN + grp_ref[...]
        seg_iota = lax.broadcasted_iota(jnp.int32, (n_seg, _N), 0)
        oht = (seg_iota == comb).astype(jnp.float32)
        gsum = jnp.dot(oht, hv_v, preferred_element_type=jnp.float32)
        cnt = jnp.sum(oht, axis=1, keepdims=True)
        means = jnp.where(cnt > 0.0, gsum / jnp.maximum(cnt, 1.0), 0.0)
        cntpos = (cnt > 0.0).astype(jnp.float32)

        row_iota = lax.broadcasted_iota(jnp.int32, (_B, n_seg), 0)
        col_iota = lax.broadcasted_iota(jnp.int32, (_B, n_seg), 1)

        def step_sel(t):
            return (col_iota == row_iota * _MAX_LEN + t).astype(jnp.float32)

        acc = jnp.zeros((_B, _HID), dtype=jnp.float32)
        h = jnp.zeros((_B, _HID), dtype=jnp.float32)
        for t in range(_MAX_LEN):
            sel = step_sel(t)
            x = jnp.dot(sel, means, preferred_element_type=jnp.float32)
            h = jnp.tanh(
                jnp.dot(x, wihf_ref[...], preferred_element_type=jnp.float32)
                + jnp.dot(h, whhf_ref[...], preferred_element_type=jnp.float32)
                + bf_ref[...])
            mask = jnp.dot(sel, cntpos, preferred_element_type=jnp.float32)
            acc = acc + 0.5 * h * mask
        h = jnp.zeros((_B, _HID), dtype=jnp.float32)
        for t in range(_MAX_LEN - 1, -1, -1):
            sel = step_sel(t)
            x = jnp.dot(sel, means, preferred_element_type=jnp.float32)
            h = jnp.tanh(
                jnp.dot(x, wihb_ref[...], preferred_element_type=jnp.float32)
                + jnp.dot(h, whhb_ref[...], preferred_element_type=jnp.float32)
                + bb_ref[...])
            mask = jnp.dot(sel, cntpos, preferred_element_type=jnp.float32)
            acc = acc + 0.5 * h * mask

        gid_iota = lax.broadcasted_iota(jnp.int32, (_B, _N), 0)
        ohg = (gid_iota == gid_ref[...]).astype(jnp.float32)
        gp = jnp.dot(ohg, hv_v, preferred_element_type=jnp.float32)
        gcnt = jnp.sum(ohg, axis=1, keepdims=True)
        gpool = gp / jnp.maximum(gcnt, 1.0)

        out_ref[...] = (
            jnp.dot(acc, wo1_ref[...], preferred_element_type=jnp.float32)
            + jnp.dot(gpool, wo2_ref[...], preferred_element_type=jnp.float32)
            + bo_ref[...])

    return pl.pallas_call(
        body,
        out_shape=jax.ShapeDtypeStruct((_B, _HID), jnp.float32),
    )(hv, gid_row, grp_row, wih_f, whh_f, bsum_f, wih_b, whh_b, bsum_b,
      wo1, wo2, bout_row)


def kernel(edge_index, node_cat, edge_cat, graph_ids, group_ids, node_emb,
           edge_emb, bn_w, bn_b, lin_W, lin_b, beta, W_ih, W_hh, b_ih, b_hh,
           W_out, b_out):
    src = edge_index[0]
    dst = edge_index[1]
    npad = _EPAD - _E

    src_pad = jnp.concatenate([src, jnp.zeros((npad,), jnp.int32)])
    dst_pad = jnp.concatenate([dst, jnp.full((npad,), _N, jnp.int32)])
    src3 = src_pad.reshape(_NW, _CPW, _CHUNK)
    dst3 = dst_pad.reshape(_NW, _CPW, _CHUNK)
    cat_col = jnp.concatenate(
        [edge_cat, jnp.zeros((npad,), jnp.int32)]
    ).astype(jnp.float32).reshape(_EPAD, 1)

    zeros_acc = jnp.zeros((_ACC_ROWS, _HID), dtype=jnp.float32)
    ntab_pad = jnp.concatenate(
        [node_emb, jnp.zeros((128 - node_emb.shape[0], _HID))]).astype(
            jnp.float32)
    etab_pad = jnp.concatenate(
        [edge_emb, jnp.zeros((16 - edge_emb.shape[0], _HID))]).astype(
            jnp.float32)

    hv = _tc_embed(node_cat.astype(jnp.float32).reshape(_N, 1), ntab_pad)

    cat_pad = jnp.concatenate([edge_cat, jnp.zeros((npad,), jnp.int32)])
    pidx_flat = _sc_pidx(node_cat, src_pad, cat_pad)

    bn_w_row = bn_w[0].reshape(1, _HID)
    bn_b_row = bn_b[0].reshape(1, _HID)
    beta_row = jnp.broadcast_to(beta[0], (1, _HID)).astype(jnp.float32)
    hv1, ex_tab, w_tab = _tc_tab0(hv, ntab_pad, etab_pad,
                                  bn_w_row, bn_b_row, beta_row)
    s_part, w_part = _sc_scatter_tab0(ex_tab, w_tab, pidx_flat, dst_pad,
                                      zeros_acc)
    hv = _tc_combine_linear(s_part, w_part, hv1, hv,
                            lin_W[0].T, lin_b[0].reshape(1, _HID))

    for l in range(1, _LAYERS):
        bn_w_row = bn_w[l].reshape(1, _HID)
        bn_b_row = bn_b[l].reshape(1, _HID)
        beta_row = jnp.broadcast_to(beta[l], (1, _HID)).astype(jnp.float32)
        hv1, m_row = _tc_bn_relu(hv, bn_w_row, bn_b_row, beta_row, edge_emb)
        g = _sc_gather_rows(hv1, src3)
        ex, w = _tc_edge_elem(g, cat_col, m_row, beta_row, etab_pad)
        s_part, w_part = _sc_scatter_add2(ex, w, dst3, zeros_acc)
        hv = _tc_combine_linear(s_part, w_part, hv1, hv,
                                lin_W[l].T, lin_b[l].reshape(1, _HID))

    gid_row = graph_ids.reshape(1, _N)
    grp_row = group_ids.reshape(1, _N)
    bsum_f = (b_ih[0] + b_hh[0]).reshape(1, _HID)
    bsum_b = (b_ih[1] + b_hh[1]).reshape(1, _HID)
    return _tc_head(hv, gid_row, grp_row,
                    W_ih[0].T, W_hh[0].T, bsum_f,
                    W_ih[1].T, W_hh[1].T, bsum_b,
                    W_out[:, :_HID].T, W_out[:, _HID:].T,
                    b_out.reshape(1, _HID))
